# baseline (device time: 56963 ns/iter reference)
import jax
import jax.numpy as jnp
from jax import lax
from jax.experimental import pallas as pl
from jax.experimental.pallas import tpu as pltpu

B, SQ, H, D = 2, 512, 8, 64
SCALE = D ** -0.5
LOG2E = 1.4426950408889634


def kernel(Q, K, V):
    def body(q_hbm, k_hbm, v_hbm, out_hbm,
             q_st, k_st, v_st, out_st,
             q2d, kl, vl, kr, vr,
             dma_sems, send_sems, recv_sems):
        my_x = lax.axis_index("x")
        my_y = lax.axis_index("y")
        my_z = lax.axis_index("z")
        partner = (1 - my_x, my_y, my_z)

        cp_k = pltpu.make_async_copy(k_hbm, k_st, dma_sems.at[1])
        cp_v = pltpu.make_async_copy(v_hbm, v_st, dma_sems.at[2])
        cp_q = pltpu.make_async_copy(q_hbm, q_st, dma_sems.at[0])
        cp_k.start()
        cp_v.start()
        cp_q.start()

        barrier = pltpu.get_barrier_semaphore()
        pl.semaphore_signal(barrier, inc=1, device_id=partner,
                            device_id_type=pl.DeviceIdType.MESH)
        pl.semaphore_wait(barrier, 1)

        cp_k.wait()
        kl[...] = k_st[...].reshape(B * SQ, H * D).astype(jnp.bfloat16)
        cp_v.wait()
        vl[...] = v_st[...].reshape(B * SQ, H * D).astype(jnp.bfloat16)

        rdma_k = pltpu.make_async_remote_copy(
            src_ref=kl, dst_ref=kr,
            send_sem=send_sems.at[0], recv_sem=recv_sems.at[0],
            device_id=partner, device_id_type=pl.DeviceIdType.MESH)
        rdma_v = pltpu.make_async_remote_copy(
            src_ref=vl, dst_ref=vr,
            send_sem=send_sems.at[1], recv_sem=recv_sems.at[1],
            device_id=partner, device_id_type=pl.DeviceIdType.MESH)
        rdma_k.start()
        rdma_v.start()

        cp_q.wait()
        q2d[...] = (q_st[...].reshape(B * SQ, H * D)
                    * (SCALE * LOG2E)).astype(jnp.bfloat16)

        ones8 = jnp.ones((SQ, 8), jnp.bfloat16)

        def partial(q_h, kv2d, b, h):
            ks = kv2d[0][b * SQ:(b + 1) * SQ, h * D:(h + 1) * D]
            vs = kv2d[1][b * SQ:(b + 1) * SQ, h * D:(h + 1) * D]
            s = lax.dot_general(q_h, ks, (((1,), (1,)), ((), ())),
                                preferred_element_type=jnp.float32)
            p = jnp.exp2(s.astype(jnp.bfloat16))
            l8 = lax.dot_general(p, ones8, (((1,), (0,)), ((), ())),
                                 preferred_element_type=jnp.float32)
            o = lax.dot_general(p, vs, (((1,), (0,)), ((), ())),
                                preferred_element_type=jnp.float32)
            return o, l8[:, :1]

        q_all = q2d[...]
        k_loc, v_loc = kl[...], vl[...]
        partials = []
        for b in range(B):
            q_b = q_all[b * SQ:(b + 1) * SQ, :]
            for h in range(H):
                partials.append(
                    partial(q_b[:, h * D:(h + 1) * D], (k_loc, v_loc), b, h))

        rdma_k.wait()
        rdma_v.wait()

        k_rem, v_rem = kr[...], vr[...]
        for b in range(B):
            q_b = q_all[b * SQ:(b + 1) * SQ, :]
            outs = []
            for h in range(H):
                o1, l1 = partials[b * H + h]
                o2, l2 = partial(q_b[:, h * D:(h + 1) * D],
                                 (k_rem, v_rem), b, h)
                outs.append((o1 + o2) / (l1 + l2))
            out_st[b] = jnp.concatenate(outs, axis=1).reshape(SQ, H, D)

        cp_out = pltpu.make_async_copy(out_st, out_hbm, dma_sems.at[3])
        cp_out.start()
        cp_out.wait()

    return pl.pallas_call(
        body,
        out_shape=jax.ShapeDtypeStruct((B, SQ, H, D), jnp.float32),
        in_specs=[pl.BlockSpec(memory_space=pl.ANY)] * 3,
        out_specs=pl.BlockSpec(memory_space=pl.ANY),
        scratch_shapes=[
            pltpu.VMEM((B, SQ, H, D), jnp.float32),
            pltpu.VMEM((B, SQ, H, D), jnp.float32),
            pltpu.VMEM((B, SQ, H, D), jnp.float32),
            pltpu.VMEM((B, SQ, H, D), jnp.float32),
            pltpu.VMEM((B * SQ, H * D), jnp.bfloat16),
            pltpu.VMEM((B * SQ, H * D), jnp.bfloat16),
            pltpu.VMEM((B * SQ, H * D), jnp.bfloat16),
            pltpu.VMEM((B * SQ, H * D), jnp.bfloat16),
            pltpu.VMEM((B * SQ, H * D), jnp.bfloat16),
            pltpu.SemaphoreType.DMA((4,)),
            pltpu.SemaphoreType.DMA((2,)),
            pltpu.SemaphoreType.DMA((2,)),
        ],
        compiler_params=pltpu.CompilerParams(
            collective_id=0, vmem_limit_bytes=100 * 1024 * 1024),
    )(Q, K, V)
